# baseline (speedup 1.0000x reference)
import functools

import jax
import jax.numpy as jnp
from jax import lax
from jax.experimental import pallas as pl
from jax.experimental.pallas import tpu as pltpu
from jax.experimental.pallas import tpu_sc as plsc

_N = 10000
_A = 32
_E = 128
_P625 = 625
_HTAB = 1024
_TILE = 1000
_GRID = _N // _TILE

_NHI = 17
_RPW_HI = 320
_RPW_LO = 304
_SPLIT = _NHI * _RPW_HI


def _tc_body(anchor_ref, d_ref, e_ref, w_ref, b_ref, m1_ref, h_ref,
             p_scr, dt_scr):
    t = pl.program_id(0)

    @pl.when(t == 0)
    def _init():
        E = e_ref[...]
        W1 = w_ref[:, :_E]
        W2 = w_ref[:, _E:]
        B2 = e_ref[...].reshape(_P625, 16, _E).sum(axis=1)
        r_io = lax.broadcasted_iota(jnp.int32, (_HTAB, _P625), 0)
        j_io = lax.broadcasted_iota(jnp.int32, (_HTAB, _P625), 1)
        perm = (jnp.equal((2 * r_io) % _P625, j_io)
                | jnp.equal((2 * r_io + 1) % _P625, j_io)).astype(jnp.float32)
        S2 = jnp.dot(perm, B2, preferred_element_type=jnp.float32)
        h_ref[...] = lax.dot_general(
            S2, W2, (((1,), (1,)), ((), ())),
            preferred_element_type=jnp.float32) * (1.0 / _A)
        rows = lax.broadcasted_iota(jnp.int32, (_A, _N), 1)
        oh = jnp.equal(anchor_ref[...], rows).astype(jnp.float32)
        A32 = jnp.dot(oh, E, preferred_element_type=jnp.float32)
        p_scr[...] = lax.dot_general(
            A32, W1, (((1,), (1,)), ((), ())),
            preferred_element_type=jnp.float32) * (1.0 / _A)
        dt_scr[...] = jnp.transpose(d_ref[...], (1, 0))

    m1_ref[...] = (jnp.dot(dt_scr[pl.ds(t * _TILE, _TILE), :], p_scr[...],
                           preferred_element_type=jnp.float32) + b_ref[...])


def _tc_stage(anchor2d, dists, embeds, W, b2d):
    return pl.pallas_call(
        _tc_body,
        grid=(_GRID,),
        in_specs=[
            pl.BlockSpec((_A, 1), lambda t: (0, 0)),
            pl.BlockSpec((_A, _N), lambda t: (0, 0)),
            pl.BlockSpec((_N, _E), lambda t: (0, 0)),
            pl.BlockSpec((_E, 2 * _E), lambda t: (0, 0)),
            pl.BlockSpec((1, _E), lambda t: (0, 0)),
        ],
        out_specs=[
            pl.BlockSpec((_TILE, _E), lambda t: (t, 0)),
            pl.BlockSpec((_HTAB, _E), lambda t: (0, 0)),
        ],
        out_shape=[
            jax.ShapeDtypeStruct((_N, _E), jnp.float32),
            jax.ShapeDtypeStruct((_HTAB, _E), jnp.float32),
        ],
        scratch_shapes=[
            pltpu.VMEM((_A, _E), jnp.float32),
            pltpu.VMEM((_N, _A), jnp.float32),
        ],
    )(anchor2d, dists, embeds, W, b2d)


def _sc_body(m1_hbm, h_hbm, out_hbm, h_v, m1_v, sem):
    c = lax.axis_index("c")
    s = lax.axis_index("s")
    wid = s * 2 + c

    def run(base, cnt):
        base = pl.multiple_of(base, 8)
        s_w = lax.rem(base, _P625)
        d = lax.rem(s_w, 8)
        s8 = pl.multiple_of(s_w - d, 8)
        cp_m = pltpu.async_copy(m1_hbm.at[pl.ds(base, cnt)],
                                m1_v.at[pl.ds(0, cnt)], sem)
        cp_h = pltpu.async_copy(h_hbm.at[pl.ds(s8, cnt + 8)],
                                h_v.at[pl.ds(0, cnt + 8)], sem)
        cp_m.wait()
        cp_h.wait()

        def body(r, carry):
            for q in range(_E // 16):
                sl = pl.ds(q * 16, 16)
                plsc.addupdate(m1_v.at[r, sl], h_v[r + d, sl])
            return carry

        lax.fori_loop(0, cnt, body, 0)
        pltpu.sync_copy(m1_v.at[pl.ds(0, cnt)], out_hbm.at[pl.ds(base, cnt)])

    @pl.when(wid < _NHI)
    def _hi():
        run(wid * _RPW_HI, _RPW_HI)

    @pl.when(wid >= _NHI)
    def _lo():
        run(_SPLIT + (wid - _NHI) * _RPW_LO, _RPW_LO)


def _sc_expand(m1, h):
    sc_kernel = functools.partial(
        pl.kernel,
        mesh=plsc.VectorSubcoreMesh(core_axis_name="c", subcore_axis_name="s"),
        out_type=jax.ShapeDtypeStruct((_N, _E), jnp.float32),
        scratch_types=[
            pltpu.VMEM((_RPW_HI + 8, _E), jnp.float32),
            pltpu.VMEM((_RPW_HI, _E), jnp.float32),
            pltpu.SemaphoreType.DMA,
        ],
    )(_sc_body)
    return sc_kernel(m1, h)


def kernel(anchor_set_id, dists_array, embeds, W, b):
    anchor2d = anchor_set_id.reshape(_A, 1)
    b2d = b.reshape(1, _E)
    m1, h = _tc_stage(anchor2d, dists_array, embeds, W, b2d)
    return _sc_expand(m1, h)


# speedup vs baseline: 1.0160x; 1.0160x over previous
import functools

import jax
import jax.numpy as jnp
from jax import lax
from jax.experimental import pallas as pl
from jax.experimental.pallas import tpu as pltpu
from jax.experimental.pallas import tpu_sc as plsc

_N = 10000
_A = 32
_E = 128
_P625 = 625
_HTAB = 1024
_TILE = 1000
_GRID = _N // _TILE

_NHI = 17
_RPW_HI = 320
_RPW_LO = 304
_SPLIT = _NHI * _RPW_HI


def _tc_body(anchor_ref, d_ref, e_ref, w_ref, b_ref, m1_ref, h_ref,
             p_scr, dt_scr):
    t = pl.program_id(0)

    @pl.when(t == 0)
    def _init():
        E = e_ref[...]
        W1 = w_ref[:, :_E]
        W2 = w_ref[:, _E:]
        B2 = e_ref[...].reshape(_P625, 16, _E).sum(axis=1)
        r_io = lax.broadcasted_iota(jnp.int32, (_HTAB, _P625), 0)
        j_io = lax.broadcasted_iota(jnp.int32, (_HTAB, _P625), 1)
        perm = (jnp.equal((2 * r_io) % _P625, j_io)
                | jnp.equal((2 * r_io + 1) % _P625, j_io)).astype(jnp.float32)
        S2 = jnp.dot(perm, B2, preferred_element_type=jnp.float32)
        h_ref[...] = lax.dot_general(
            S2, W2, (((1,), (1,)), ((), ())),
            preferred_element_type=jnp.float32) * (1.0 / _A)
        rows = lax.broadcasted_iota(jnp.int32, (_A, _N), 1)
        oh = jnp.equal(anchor_ref[...], rows).astype(jnp.float32)
        A32 = jnp.dot(oh, E, preferred_element_type=jnp.float32)
        p_scr[...] = lax.dot_general(
            A32, W1, (((1,), (1,)), ((), ())),
            preferred_element_type=jnp.float32) * (1.0 / _A)
        dt_scr[...] = jnp.transpose(d_ref[...], (1, 0))

    m1_ref[...] = (jnp.dot(dt_scr[pl.ds(t * _TILE, _TILE), :], p_scr[...],
                           preferred_element_type=jnp.float32) + b_ref[...])


def _tc_stage(anchor2d, dists, embeds, W, b2d):
    return pl.pallas_call(
        _tc_body,
        grid=(_GRID,),
        in_specs=[
            pl.BlockSpec((_A, 1), lambda t: (0, 0)),
            pl.BlockSpec((_A, _N), lambda t: (0, 0)),
            pl.BlockSpec((_N, _E), lambda t: (0, 0)),
            pl.BlockSpec((_E, 2 * _E), lambda t: (0, 0)),
            pl.BlockSpec((1, _E), lambda t: (0, 0)),
        ],
        out_specs=[
            pl.BlockSpec((_TILE, _E), lambda t: (t, 0)),
            pl.BlockSpec((_HTAB, _E), lambda t: (0, 0)),
        ],
        out_shape=[
            jax.ShapeDtypeStruct((_N, _E), jnp.float32),
            jax.ShapeDtypeStruct((_HTAB, _E), jnp.float32),
        ],
        scratch_shapes=[
            pltpu.VMEM((_A, _E), jnp.float32),
            pltpu.VMEM((_N, _A), jnp.float32),
        ],
    )(anchor2d, dists, embeds, W, b2d)


def _sc_body(m1_hbm, h_hbm, out_hbm, h_v, m1_v,
             hsem, wsem, msem0, msem1, msem2, msem3):
    c = lax.axis_index("c")
    s = lax.axis_index("s")
    wid = s * 2 + c
    msems = [msem0, msem1, msem2, msem3]

    def run(base, cnt, chunks):
        base = pl.multiple_of(base, 8)
        s_w = lax.rem(base, _P625)
        d = lax.rem(s_w, 8)
        s8 = pl.multiple_of(s_w - d, 8)
        cp_h = pltpu.async_copy(h_hbm.at[pl.ds(s8, cnt + 8)],
                                h_v.at[pl.ds(0, cnt + 8)], hsem)
        cps = []
        off = 0
        for i, csz in enumerate(chunks):
            cps.append((off, csz, pltpu.async_copy(
                m1_hbm.at[pl.ds(base + off, csz)],
                m1_v.at[pl.ds(off, csz)], msems[i])))
            off += csz
        cp_h.wait()
        writes = []
        for off, csz, cp in cps:
            cp.wait()

            def body(r2, carry, _off=off):
                for u in range(2):
                    r = r2 * 2 + u + _off
                    for q in range(_E // 16):
                        sl = pl.ds(q * 16, 16)
                        plsc.addupdate(m1_v.at[r, sl], h_v[r + d, sl])
                return carry

            lax.fori_loop(0, csz // 2, body, 0)
            writes.append(pltpu.async_copy(
                m1_v.at[pl.ds(off, csz)],
                out_hbm.at[pl.ds(base + off, csz)], wsem))
        for w in writes:
            w.wait()

    @pl.when(wid < _NHI)
    def _hi():
        run(wid * _RPW_HI, _RPW_HI, (80, 80, 80, 80))

    @pl.when(wid >= _NHI)
    def _lo():
        run(_SPLIT + (wid - _NHI) * _RPW_LO, _RPW_LO, (80, 80, 80, 64))


def _sc_expand(m1, h):
    sc_kernel = functools.partial(
        pl.kernel,
        mesh=plsc.VectorSubcoreMesh(core_axis_name="c", subcore_axis_name="s"),
        out_type=jax.ShapeDtypeStruct((_N, _E), jnp.float32),
        scratch_types=[
            pltpu.VMEM((_RPW_HI + 8, _E), jnp.float32),
            pltpu.VMEM((_RPW_HI, _E), jnp.float32),
            pltpu.SemaphoreType.DMA,
            pltpu.SemaphoreType.DMA,
            pltpu.SemaphoreType.DMA,
            pltpu.SemaphoreType.DMA,
            pltpu.SemaphoreType.DMA,
            pltpu.SemaphoreType.DMA,
        ],
    )(_sc_body)
    return sc_kernel(m1, h)


def kernel(anchor_set_id, dists_array, embeds, W, b):
    anchor2d = anchor_set_id.reshape(_A, 1)
    b2d = b.reshape(1, _E)
    m1, h = _tc_stage(anchor2d, dists_array, embeds, W, b2d)
    return _sc_expand(m1, h)


# speedup vs baseline: 1.1539x; 1.1357x over previous
import functools

import jax
import jax.numpy as jnp
from jax import lax
from jax.experimental import pallas as pl
from jax.experimental.pallas import tpu as pltpu
from jax.experimental.pallas import tpu_sc as plsc

_N = 10000
_A = 32
_E = 128
_P625 = 625
_HTAB = 1024
_TILE = 1000
_GRID = _N // _TILE

_NHI = 17
_RPW_HI = 320
_RPW_LO = 304
_SPLIT = _NHI * _RPW_HI


def _tc_body(anchor_ref, d_ref, e_ref, w_ref, b_ref, m1_ref, h_ref,
             p_scr, dt_scr):
    t = pl.program_id(0)

    @pl.when(t == 0)
    def _init():
        E = e_ref[...]
        W1 = w_ref[:, :_E]
        W2 = w_ref[:, _E:]
        B2 = e_ref[...].reshape(_P625, 16, _E).sum(axis=1)
        r_io = lax.broadcasted_iota(jnp.int32, (_HTAB, _P625), 0)
        j_io = lax.broadcasted_iota(jnp.int32, (_HTAB, _P625), 1)
        perm = (jnp.equal((2 * r_io) % _P625, j_io)
                | jnp.equal((2 * r_io + 1) % _P625, j_io)).astype(jnp.float32)
        S2 = jnp.dot(perm, B2, preferred_element_type=jnp.float32)
        h_ref[...] = lax.dot_general(
            S2, W2, (((1,), (1,)), ((), ())),
            preferred_element_type=jnp.float32) * (1.0 / _A)
        rows = lax.broadcasted_iota(jnp.int32, (_A, _N), 1)
        oh = jnp.equal(anchor_ref[...], rows).astype(jnp.float32)
        A32 = jnp.dot(oh, E, preferred_element_type=jnp.float32)
        p_scr[...] = lax.dot_general(
            A32, W1, (((1,), (1,)), ((), ())),
            preferred_element_type=jnp.float32) * (1.0 / _A)
        dt_scr[...] = jnp.transpose(d_ref[...], (1, 0))

    m1_ref[...] = (jnp.dot(dt_scr[pl.ds(t * _TILE, _TILE), :], p_scr[...],
                           preferred_element_type=jnp.float32) + b_ref[...])


def _tc_stage(anchor2d, dists, embeds, W, b2d):
    return pl.pallas_call(
        _tc_body,
        grid=(_GRID,),
        in_specs=[
            pl.BlockSpec((_A, 1), lambda t: (0, 0)),
            pl.BlockSpec((_A, _N), lambda t: (0, 0)),
            pl.BlockSpec((_N, _E), lambda t: (0, 0)),
            pl.BlockSpec((_E, 2 * _E), lambda t: (0, 0)),
            pl.BlockSpec((1, _E), lambda t: (0, 0)),
        ],
        out_specs=[
            pl.BlockSpec((_TILE, _E), lambda t: (t, 0)),
            pl.BlockSpec((_HTAB, _E), lambda t: (0, 0)),
        ],
        out_shape=[
            jax.ShapeDtypeStruct((_N, _E), jnp.float32),
            jax.ShapeDtypeStruct((_HTAB, _E), jnp.float32),
        ],
        scratch_shapes=[
            pltpu.VMEM((_A, _E), jnp.float32),
            pltpu.VMEM((_N, _A), jnp.float32),
        ],
    )(anchor2d, dists, embeds, W, b2d)


def _sc_body(m1_hbm, h_hbm, out_hbm, h_v, m1_v,
             hsem, wsem, msem0, msem1, msem2, msem3):
    c = lax.axis_index("c")
    s = lax.axis_index("s")
    wid = s * 2 + c
    msems = [msem0, msem1, msem2, msem3]

    def run(base, cnt, chunks):
        base = pl.multiple_of(base, 8)
        s_w = lax.rem(base, _P625)
        d = lax.rem(s_w, 8)
        s8 = pl.multiple_of(s_w - d, 8)
        cp_h = pltpu.async_copy(h_hbm.at[pl.ds(s8, cnt + 8)],
                                h_v.at[pl.ds(0, cnt + 8)], hsem)
        cps = []
        off = 0
        for i, csz in enumerate(chunks):
            cps.append((off, csz, pltpu.async_copy(
                m1_hbm.at[pl.ds(base + off, csz)],
                m1_v.at[pl.ds(off, csz)], msems[i])))
            off += csz
        cp_h.wait()
        writes = []
        for off, csz, cp in cps:
            cp.wait()

            @plsc.parallel_loop(off, off + csz, unroll=8)
            def body(r):
                for q in range(_E // 16):
                    sl = pl.ds(q * 16, 16)
                    plsc.addupdate(m1_v.at[r, sl], h_v[r + d, sl])
            writes.append(pltpu.async_copy(
                m1_v.at[pl.ds(off, csz)],
                out_hbm.at[pl.ds(base + off, csz)], wsem))
        for w in writes:
            w.wait()

    @pl.when(wid < _NHI)
    def _hi():
        run(wid * _RPW_HI, _RPW_HI, (80, 80, 80, 80))

    @pl.when(wid >= _NHI)
    def _lo():
        run(_SPLIT + (wid - _NHI) * _RPW_LO, _RPW_LO, (80, 80, 80, 64))


def _sc_expand(m1, h):
    sc_kernel = functools.partial(
        pl.kernel,
        mesh=plsc.VectorSubcoreMesh(core_axis_name="c", subcore_axis_name="s"),
        out_type=jax.ShapeDtypeStruct((_N, _E), jnp.float32),
        scratch_types=[
            pltpu.VMEM((_RPW_HI + 8, _E), jnp.float32),
            pltpu.VMEM((_RPW_HI, _E), jnp.float32),
            pltpu.SemaphoreType.DMA,
            pltpu.SemaphoreType.DMA,
            pltpu.SemaphoreType.DMA,
            pltpu.SemaphoreType.DMA,
            pltpu.SemaphoreType.DMA,
            pltpu.SemaphoreType.DMA,
        ],
    )(_sc_body)
    return sc_kernel(m1, h)


def kernel(anchor_set_id, dists_array, embeds, W, b):
    anchor2d = anchor_set_id.reshape(_A, 1)
    b2d = b.reshape(1, _E)
    m1, h = _tc_stage(anchor2d, dists_array, embeds, W, b2d)
    return _sc_expand(m1, h)


# speedup vs baseline: 1.4779x; 1.2808x over previous
import functools

import jax
import jax.numpy as jnp
from jax import lax
from jax.experimental import pallas as pl
from jax.experimental.pallas import tpu as pltpu
from jax.experimental.pallas import tpu_sc as plsc

_N = 10000
_A = 32
_E = 128
_P625 = 625
_TILE = 1000
_GRID = _N // _TILE
_HTAB = _P625 + _TILE


def _sc_gather_body(anchor_hbm, emb_hbm, a32_hbm, idx_v, rows_v, sem):
    c = lax.axis_index("c")
    s = lax.axis_index("s")
    wid = s * 2 + c

    @pl.when(wid == 0)
    def _():
        pltpu.sync_copy(anchor_hbm, idx_v)
        pltpu.async_copy(emb_hbm.at[idx_v], rows_v, sem).wait()
        pltpu.sync_copy(rows_v, a32_hbm)


def _sc_gather(anchor_set_id, embeds):
    sc_kernel = functools.partial(
        pl.kernel,
        mesh=plsc.VectorSubcoreMesh(core_axis_name="c", subcore_axis_name="s"),
        out_type=jax.ShapeDtypeStruct((_A, _E), jnp.float32),
        scratch_types=[
            pltpu.VMEM((_A,), jnp.int32),
            pltpu.VMEM((_A, _E), jnp.float32),
            pltpu.SemaphoreType.DMA,
        ],
    )(_sc_gather_body)
    return sc_kernel(anchor_set_id, embeds)


def _tc_body(a32_ref, d_ref, e_ref, w_ref, b_ref, out_ref,
             p_scr, h_scr, dt_scr):
    t = pl.program_id(0)

    @pl.when(t == 0)
    def _init():
        W1 = w_ref[:, :_E]
        W2 = w_ref[:, _E:]
        B2 = e_ref[...].reshape(_P625, 16, _E).sum(axis=1)
        r_io = lax.broadcasted_iota(jnp.int32, (_HTAB, _P625), 0)
        j_io = lax.broadcasted_iota(jnp.int32, (_HTAB, _P625), 1)
        perm = (jnp.equal((2 * r_io) % _P625, j_io)
                | jnp.equal((2 * r_io + 1) % _P625, j_io)).astype(jnp.float32)
        S2 = jnp.dot(perm, B2, preferred_element_type=jnp.float32)
        h_scr[...] = lax.dot_general(
            S2, W2, (((1,), (1,)), ((), ())),
            preferred_element_type=jnp.float32) * (1.0 / _A) + b_ref[...]
        p_scr[...] = lax.dot_general(
            a32_ref[...], W1, (((1,), (1,)), ((), ())),
            preferred_element_type=jnp.float32) * (1.0 / _A)
        dt_scr[...] = jnp.transpose(d_ref[...], (1, 0))

    s_t = (t * _TILE) % _P625
    out_ref[...] = (jnp.dot(dt_scr[pl.ds(t * _TILE, _TILE), :], p_scr[...],
                            preferred_element_type=jnp.float32)
                    + h_scr[pl.ds(s_t, _TILE), :])


def kernel(anchor_set_id, dists_array, embeds, W, b):
    a32 = _sc_gather(anchor_set_id, embeds)
    b2d = b.reshape(1, _E)
    return pl.pallas_call(
        _tc_body,
        grid=(_GRID,),
        in_specs=[
            pl.BlockSpec((_A, _E), lambda t: (0, 0)),
            pl.BlockSpec((_A, _N), lambda t: (0, 0)),
            pl.BlockSpec((_N, _E), lambda t: (0, 0)),
            pl.BlockSpec((_E, 2 * _E), lambda t: (0, 0)),
            pl.BlockSpec((1, _E), lambda t: (0, 0)),
        ],
        out_specs=pl.BlockSpec((_TILE, _E), lambda t: (t, 0)),
        out_shape=jax.ShapeDtypeStruct((_N, _E), jnp.float32),
        scratch_shapes=[
            pltpu.VMEM((_A, _E), jnp.float32),
            pltpu.VMEM((_HTAB, _E), jnp.float32),
            pltpu.VMEM((_N, _A), jnp.float32),
        ],
    )(a32, dists_array, embeds, W, b2d)
